# baseline (speedup 1.0000x reference)
import functools

import jax
import jax.numpy as jnp
from jax import lax
from jax.experimental import pallas as pl
from jax.experimental.pallas import tpu as pltpu
from jax.experimental.pallas import tpu_sc as plsc

BATCH = 4096
SEQ = 200
D = 64
N = BATCH * SEQ
NW = 32
PER_W = N // NW
CH = 512
NCH = PER_W // CH


def _make_gather():
  mesh = plsc.VectorSubcoreMesh(core_axis_name="c", subcore_axis_name="s")

  @functools.partial(
      pl.kernel,
      mesh=mesh,
      out_type=jax.ShapeDtypeStruct((N, D), jnp.float32),
      scratch_types=[
          pltpu.VMEM((CH,), jnp.int32),
          pltpu.VMEM((CH, D), jnp.float32),
          pltpu.SemaphoreType.DMA,
      ],
      compiler_params=pltpu.CompilerParams(use_tc_tiling_on_sc=False),
  )
  def k(idx_hbm, table_hbm, out_hbm, idx_v, rows_v, sem):
    wid = lax.axis_index("s") * 2 + lax.axis_index("c")
    base = wid * PER_W

    def body(g, carry):
      off = base + g * CH
      pltpu.sync_copy(idx_hbm.at[pl.ds(off, CH)], idx_v)
      pltpu.async_copy(table_hbm.at[idx_v], rows_v, sem).wait()
      pltpu.sync_copy(rows_v, out_hbm.at[pl.ds(off, CH)])
      return carry

    lax.fori_loop(0, NCH, body, 0)

  return k


_gather = _make_gather()


@jax.jit
def kernel(idx, table):
  flat = idx.reshape(N).astype(jnp.int32)
  out = _gather(flat, table)
  return out.reshape(BATCH, SEQ, D)


# speedup vs baseline: 1.0455x; 1.0455x over previous
import functools

import jax
import jax.numpy as jnp
from jax import lax
from jax.experimental import pallas as pl
from jax.experimental.pallas import tpu as pltpu
from jax.experimental.pallas import tpu_sc as plsc

BATCH = 4096
SEQ = 200
D = 64
N = BATCH * SEQ
NW = 32
PER_W = N // NW
CH = 512
NCH = PER_W // CH
NBUF = 2


def _make_gather():
  mesh = plsc.VectorSubcoreMesh(core_axis_name="c", subcore_axis_name="s")

  @functools.partial(
      pl.kernel,
      mesh=mesh,
      out_type=jax.ShapeDtypeStruct((N, D), jnp.float32),
      scratch_types=[
          pltpu.VMEM((NCH, CH), jnp.int32),
          pltpu.VMEM((CH, D), jnp.float32),
          pltpu.VMEM((CH, D), jnp.float32),
          pltpu.SemaphoreType.DMA,
          pltpu.SemaphoreType.DMA,
          pltpu.SemaphoreType.DMA,
          pltpu.SemaphoreType.DMA,
      ],
      compiler_params=pltpu.CompilerParams(use_tc_tiling_on_sc=False),
  )
  def k(idx_hbm, table_hbm, out_hbm, idx_all, rows0, rows1,
        gsem0, gsem1, ssem0, ssem1):
    wid = lax.axis_index("s") * 2 + lax.axis_index("c")
    base = wid * PER_W
    rows = (rows0, rows1)
    gsem = (gsem0, gsem1)
    ssem = (ssem0, ssem1)

    pltpu.sync_copy(idx_hbm.at[wid], idx_all)

    def start_gather(g, b):
      pltpu.async_copy(table_hbm.at[idx_all.at[g]], rows[b], gsem[b])

    def wait_gather(g, b):
      pltpu.make_async_copy(table_hbm.at[idx_all.at[g]], rows[b],
                            gsem[b]).wait()

    def start_store(g, b):
      pltpu.async_copy(rows[b], out_hbm.at[pl.ds(base + g * CH, CH)], ssem[b])

    def wait_store(g, b):
      pltpu.make_async_copy(rows[b], out_hbm.at[pl.ds(base + g * CH, CH)],
                            ssem[b]).wait()

    for b in range(NBUF):
      start_gather(b, b)

    def body(i, carry):
      for b in range(NBUF):
        g = i * NBUF + b
        wait_gather(g, b)
        start_store(g, b)
        wait_store(g, b)
        start_gather(g + NBUF, b)
      return carry

    lax.fori_loop(0, NCH // NBUF - 1, body, 0)

    for b in range(NBUF):
      g = NCH - NBUF + b
      wait_gather(g, b)
      start_store(g, b)
    for b in range(NBUF):
      wait_store(NCH - NBUF + b, b)

  return k


_gather = _make_gather()


@jax.jit
def kernel(idx, table):
  flat = idx.reshape(NW, NCH, CH).astype(jnp.int32)
  out = _gather(flat, table)
  return out.reshape(BATCH, SEQ, D)
